# baseline (speedup 1.0000x reference)
import jax
import jax.numpy as jnp
from jax.experimental import pallas as pl
from jax.experimental.pallas import tpu as pltpu

_LANE = 128
_W = 8
_CO = 8
_CI = 3


def _fused_kernel(x_ref, wc_ref, bc_ref, w2k_ref, b2k_ref, o_ref):
    bn = x_ref.shape[0]
    x0 = x_ref[:, 0 * _W:1 * _W, :]
    x1 = x_ref[:, 1 * _W:2 * _W, :]
    x2 = x_ref[:, 2 * _W:3 * _W, :]
    v = jnp.concatenate(
        [jnp.maximum(
            x0 * wc_ref[co, 0] + x1 * wc_ref[co, 1] + x2 * wc_ref[co, 2]
            + bc_ref[co, 0], 0.0)
         for co in range(_CO)], axis=1)
    w2k = w2k_ref[...]
    b2k = b2k_ref[...]
    for i in range(bn):
        o_ref[i] = jnp.dot(w2k, v[i],
                           preferred_element_type=jnp.float32) + b2k


def _pick_bn(n):
    if n <= 16:
        return max(n // 2, 1)
    return min(512, n // 2)


def kernel(x, wc, bc, w2t, b2t):
    N, C, H, W = x.shape
    Hp = pl.cdiv(H, _LANE) * _LANE
    bn = _pick_bn(N)
    Np = pl.cdiv(N, bn) * bn

    x3 = jnp.transpose(x, (0, 1, 3, 2))
    if Hp != H:
        x3 = jnp.pad(x3, ((0, 0), (0, 0), (0, 0), (0, Hp - H)))
    if Np != N:
        x3 = jnp.pad(x3, ((0, Np - N), (0, 0), (0, 0), (0, 0)))
    x3 = x3.reshape(Np, C * W, Hp)

    w2k = jnp.kron(jnp.eye(_CO, dtype=jnp.float32), w2t[:_W, :_W].T)
    b2k = jnp.broadcast_to(
        jnp.tile(b2t[0, :_W], _CO)[:, None], (_CO * _W, Hp))

    out3 = pl.pallas_call(
        _fused_kernel,
        out_shape=jax.ShapeDtypeStruct((Np, _CO * _W, Hp), jnp.float32),
        grid=(Np // bn,),
        in_specs=[
            pl.BlockSpec((bn, C * W, Hp), lambda n: (n, 0, 0)),
            pl.BlockSpec((_CO, _CI), lambda n: (0, 0)),
            pl.BlockSpec((_CO, 1), lambda n: (0, 0)),
            pl.BlockSpec((_CO * _W, _CO * _W), lambda n: (0, 0)),
            pl.BlockSpec((_CO * _W, Hp), lambda n: (0, 0)),
        ],
        out_specs=pl.BlockSpec((bn, _CO * _W, Hp), lambda n: (n, 0, 0)),
        compiler_params=pltpu.CompilerParams(
            dimension_semantics=("parallel",)),
    )(x3, wc, bc, w2k, b2k)

    out4 = out3.reshape(Np, _CO, _W, Hp)[:N, :, :, :H]
    return jnp.transpose(out4, (0, 1, 3, 2))


# speedup vs baseline: 1.0097x; 1.0097x over previous
import jax
import jax.numpy as jnp
from jax.experimental import pallas as pl
from jax.experimental.pallas import tpu as pltpu

_LANE = 128
_W = 8
_CO = 8
_CI = 3


def _fused_kernel(x_ref, wc_ref, bc_ref, w2k_ref, b2k_ref, o_ref):
    bn = x_ref.shape[0]
    x0 = x_ref[:, 0 * _W:1 * _W, :]
    x1 = x_ref[:, 1 * _W:2 * _W, :]
    x2 = x_ref[:, 2 * _W:3 * _W, :]
    v = jnp.concatenate(
        [jnp.maximum(
            x0 * wc_ref[co, 0] + x1 * wc_ref[co, 1] + x2 * wc_ref[co, 2]
            + bc_ref[co, 0], 0.0)
         for co in range(_CO)], axis=1)
    w2k = w2k_ref[...]
    b2k = b2k_ref[...]
    for i in range(bn):
        o_ref[i] = jnp.dot(w2k, v[i],
                           preferred_element_type=jnp.float32) + b2k


def _pick_bn(n):
    if n <= 16:
        return max(n // 2, 1)
    return min(256, n // 2)


def kernel(x, wc, bc, w2t, b2t):
    N, C, H, W = x.shape
    Hp = pl.cdiv(H, _LANE) * _LANE
    bn = _pick_bn(N)
    Np = pl.cdiv(N, bn) * bn

    x3 = jnp.transpose(x, (0, 1, 3, 2))
    if Hp != H:
        x3 = jnp.pad(x3, ((0, 0), (0, 0), (0, 0), (0, Hp - H)))
    if Np != N:
        x3 = jnp.pad(x3, ((0, Np - N), (0, 0), (0, 0), (0, 0)))
    x3 = x3.reshape(Np, C * W, Hp)

    w2k = jnp.kron(jnp.eye(_CO, dtype=jnp.float32), w2t[:_W, :_W].T)
    b2k = jnp.broadcast_to(
        jnp.tile(b2t[0, :_W], _CO)[:, None], (_CO * _W, Hp))

    out3 = pl.pallas_call(
        _fused_kernel,
        out_shape=jax.ShapeDtypeStruct((Np, _CO * _W, Hp), jnp.float32),
        grid=(Np // bn,),
        in_specs=[
            pl.BlockSpec((bn, C * W, Hp), lambda n: (n, 0, 0)),
            pl.BlockSpec((_CO, _CI), lambda n: (0, 0)),
            pl.BlockSpec((_CO, 1), lambda n: (0, 0)),
            pl.BlockSpec((_CO * _W, _CO * _W), lambda n: (0, 0)),
            pl.BlockSpec((_CO * _W, Hp), lambda n: (0, 0)),
        ],
        out_specs=pl.BlockSpec((bn, _CO * _W, Hp), lambda n: (n, 0, 0)),
        compiler_params=pltpu.CompilerParams(
            dimension_semantics=("parallel",)),
    )(x3, wc, bc, w2k, b2k)

    out4 = out3.reshape(Np, _CO, _W, Hp)[:N, :, :, :H]
    return jnp.transpose(out4, (0, 1, 3, 2))


# speedup vs baseline: 1.0127x; 1.0030x over previous
import jax
import jax.numpy as jnp
from jax.experimental import pallas as pl
from jax.experimental.pallas import tpu as pltpu

_LANE = 128
_W = 8
_CO = 8
_CI = 3


def _fused_kernel(x_ref, wc_ref, bc_ref, w2k_ref, b2k_ref, o_ref):
    bn = x_ref.shape[0]
    x0 = x_ref[:, 0 * _W:1 * _W, :]
    x1 = x_ref[:, 1 * _W:2 * _W, :]
    x2 = x_ref[:, 2 * _W:3 * _W, :]
    v = jnp.concatenate(
        [jnp.maximum(
            x0 * wc_ref[co, 0] + x1 * wc_ref[co, 1] + x2 * wc_ref[co, 2]
            + bc_ref[co, 0], 0.0)
         for co in range(_CO)], axis=1)
    w2k = w2k_ref[...]
    b2k = b2k_ref[...]
    for i in range(bn):
        o_ref[i] = jnp.dot(w2k, v[i],
                           preferred_element_type=jnp.float32) + b2k


def _pick_bn(n):
    if n <= 16:
        return max(n // 2, 1)
    return min(256, n // 2)


def kernel(x, wc, bc, w2t, b2t):
    N, C, H, W = x.shape
    assert C == _CI and W == _W, "module pins C=3 (conv) and W=8 (linear2)"
    Hp = pl.cdiv(H, _LANE) * _LANE
    bn = _pick_bn(N)
    Np = pl.cdiv(N, bn) * bn

    x3 = jnp.transpose(x, (0, 1, 3, 2))
    if Hp != H:
        x3 = jnp.pad(x3, ((0, 0), (0, 0), (0, 0), (0, Hp - H)))
    if Np != N:
        x3 = jnp.pad(x3, ((0, Np - N), (0, 0), (0, 0), (0, 0)))
    x3 = x3.reshape(Np, C * W, Hp)

    w2k = jnp.kron(jnp.eye(_CO, dtype=jnp.float32), w2t[:_W, :_W].T)
    b2k = jnp.broadcast_to(
        jnp.tile(b2t[0, :_W], _CO)[:, None], (_CO * _W, Hp))

    out3 = pl.pallas_call(
        _fused_kernel,
        out_shape=jax.ShapeDtypeStruct((Np, _CO * _W, Hp), jnp.float32),
        grid=(Np // bn,),
        in_specs=[
            pl.BlockSpec((bn, C * W, Hp), lambda n: (n, 0, 0)),
            pl.BlockSpec((_CO, _CI), lambda n: (0, 0)),
            pl.BlockSpec((_CO, 1), lambda n: (0, 0)),
            pl.BlockSpec((_CO * _W, _CO * _W), lambda n: (0, 0)),
            pl.BlockSpec((_CO * _W, Hp), lambda n: (0, 0)),
        ],
        out_specs=pl.BlockSpec((bn, _CO * _W, Hp), lambda n: (n, 0, 0)),
        compiler_params=pltpu.CompilerParams(
            dimension_semantics=("parallel",)),
    )(x3, wc, bc, w2k, b2k)

    out4 = out3.reshape(Np, _CO, _W, Hp)[:N, :, :, :H]
    return jnp.transpose(out4, (0, 1, 3, 2))
